# baseline (speedup 1.0000x reference)
import dataclasses
import functools

import jax
import jax.numpy as jnp
from jax import lax
from jax.experimental import pallas as pl
from jax.experimental.pallas import tpu as pltpu
from jax.experimental.pallas import tpu_sc as plsc

NUM_ENTITIES = 1000000
NUM_RELATIONS = 1000
EMBED_DIM = 64
BATCH = 16384

NC = 2
NS = 16
NW = NC * NS
LANES = 16

B_PER_W = BATCH // NW
CHUNK = 128
N_CHUNKS = B_PER_W // CHUNK


def _trig_body(rr_ref, ri_ref, out_ref):
    rr = rr_ref[...]
    out_ref[...] = jnp.concatenate(
        [jnp.cos(rr), jnp.sin(rr), jnp.cos(ri_ref[...]),
         jnp.zeros_like(rr)], axis=1)


def _relation_trig(relation_re, relation_im):
    return pl.pallas_call(
        _trig_body,
        out_shape=jax.ShapeDtypeStruct((NUM_RELATIONS, 4 * EMBED_DIM),
                                       jnp.float32),
    )(relation_re, relation_im)


def _sc_score_kernel(h2_hbm, hp_hbm, t2_hbm, tp_hbm, r_hbm,
                     ere_hbm, eim_hbm, rel_hbm, out_hbm,
                     idx_h, idx_t, idx_r, par_h, par_t,
                     hr_v, hi_v, tr_v, ti_v, r_v, scores_v, sem):
    wid = lax.axis_index("s") * NC + lax.axis_index("c")
    base_w = wid * B_PER_W
    D = EMBED_DIM

    @pl.loop(0, N_CHUNKS)
    def _(chunk):
        base = base_w + chunk * CHUNK
        pltpu.sync_copy(h2_hbm.at[pl.ds(base, CHUNK)], idx_h)
        pltpu.sync_copy(t2_hbm.at[pl.ds(base, CHUNK)], idx_t)
        pltpu.sync_copy(r_hbm.at[pl.ds(base, CHUNK)], idx_r)
        pltpu.sync_copy(hp_hbm.at[pl.ds(base, CHUNK)], par_h)
        pltpu.sync_copy(tp_hbm.at[pl.ds(base, CHUNK)], par_t)
        copies = [
            pltpu.async_copy(ere_hbm.at[idx_h], hr_v, sem),
            pltpu.async_copy(eim_hbm.at[idx_h], hi_v, sem),
            pltpu.async_copy(ere_hbm.at[idx_t], tr_v, sem),
            pltpu.async_copy(eim_hbm.at[idx_t], ti_v, sem),
            pltpu.async_copy(rel_hbm.at[idx_r], r_v, sem),
        ]
        for cp_ in copies:
            cp_.wait()

        lane = lax.iota(jnp.int32, LANES)

        @pl.loop(0, CHUNK // LANES)
        def _(g):
            hp_vec = par_h[pl.ds(g * LANES, LANES)]
            tp_vec = par_t[pl.ds(g * LANES, LANES)]
            svec = jnp.zeros((LANES,), jnp.float32)
            for k in range(LANES):
                b = g * LANES + k
                bh = hp_vec[k] * D
                bt = tp_vec[k] * D
                acc = None
                for j in range(D // LANES):
                    o = j * LANES
                    trv = tr_v[b, pl.ds(bt + o, LANES)]
                    tiv = ti_v[b, pl.ds(bt + o, LANES)]
                    cc = r_v[b, pl.ds(o, LANES)]
                    ss = r_v[b, pl.ds(D + o, LANES)]
                    ci = r_v[b, pl.ds(2 * D + o, LANES)]
                    rot_r = trv * cc - tiv * ss
                    rot_i = trv * ss + tiv * ci
                    dr = hr_v[b, pl.ds(bh + o, LANES)] - rot_r
                    di = hi_v[b, pl.ds(bh + o, LANES)] - rot_i
                    part = dr * dr + di * di
                    acc = part if acc is None else acc + part
                svec = jnp.where(lane == k, jnp.sum(acc), svec)
            scores_v[pl.ds(g * LANES, LANES)] = svec

        pltpu.sync_copy(scores_v, out_hbm.at[pl.ds(base, CHUNK)])


def _sc_score(h2, hp, t2, tp, r, ere2, eim2, rel_tbl):
    mesh = plsc.VectorSubcoreMesh(core_axis_name="c", subcore_axis_name="s")
    cp = pltpu.CompilerParams(use_tc_tiling_on_sc=True)
    if "needs_layout_passes" in pltpu.CompilerParams.__dataclass_fields__:
        cp = dataclasses.replace(cp, needs_layout_passes=False)
    run = functools.partial(
        pl.kernel,
        mesh=mesh,
        compiler_params=cp,
        out_type=jax.ShapeDtypeStruct((BATCH,), jnp.float32),
        scratch_types=[
            pltpu.VMEM((CHUNK,), jnp.int32),
            pltpu.VMEM((CHUNK,), jnp.int32),
            pltpu.VMEM((CHUNK,), jnp.int32),
            pltpu.VMEM((CHUNK,), jnp.int32),
            pltpu.VMEM((CHUNK,), jnp.int32),
            pltpu.VMEM((CHUNK, 2 * EMBED_DIM), jnp.float32),
            pltpu.VMEM((CHUNK, 2 * EMBED_DIM), jnp.float32),
            pltpu.VMEM((CHUNK, 2 * EMBED_DIM), jnp.float32),
            pltpu.VMEM((CHUNK, 2 * EMBED_DIM), jnp.float32),
            pltpu.VMEM((CHUNK, 4 * EMBED_DIM), jnp.float32),
            pltpu.VMEM((CHUNK,), jnp.float32),
            pltpu.SemaphoreType.DMA,
        ],
    )(_sc_score_kernel)
    return run(h2, hp, t2, tp, r, ere2, eim2, rel_tbl)


def kernel(h, r, t, entity_re, entity_im, relation_re, relation_im):
    h = h.astype(jnp.int32)
    r = r.astype(jnp.int32)
    t = t.astype(jnp.int32)
    ere2 = jnp.reshape(entity_re, (NUM_ENTITIES // 2, 2 * EMBED_DIM))
    eim2 = jnp.reshape(entity_im, (NUM_ENTITIES // 2, 2 * EMBED_DIM))
    rel_tbl = _relation_trig(relation_re, relation_im)
    h2, hp = h >> 1, h & 1
    t2, tp = t >> 1, t & 1
    return _sc_score(h2, hp, t2, tp, r, ere2, eim2, rel_tbl)


# speedup vs baseline: 2.2781x; 2.2781x over previous
import dataclasses
import functools

import jax
import jax.numpy as jnp
from jax import lax
from jax.experimental import pallas as pl
from jax.experimental.pallas import tpu as pltpu
from jax.experimental.pallas import tpu_sc as plsc

NUM_ENTITIES = 1000000
NUM_RELATIONS = 1000
EMBED_DIM = 64
BATCH = 16384

NC = 2
NS = 16
NW = NC * NS
LANES = 16

B_PER_W = BATCH // NW
CHUNK = 128
N_CHUNKS = B_PER_W // CHUNK

FUSE_BLK = 8192


def _fuse_body(re_ref, im_ref, out_ref):
    out_ref[...] = jnp.concatenate(
        [jnp.transpose(re_ref[...]), jnp.transpose(im_ref[...])], axis=1)


def _fused_entity_table(entity_re, entity_im):
    re_t = jnp.transpose(entity_re)
    im_t = jnp.transpose(entity_im)
    grid = pl.cdiv(NUM_ENTITIES, FUSE_BLK)
    return pl.pallas_call(
        _fuse_body,
        grid=(grid,),
        in_specs=[
            pl.BlockSpec((EMBED_DIM, FUSE_BLK), lambda i: (0, i)),
            pl.BlockSpec((EMBED_DIM, FUSE_BLK), lambda i: (0, i)),
        ],
        out_specs=pl.BlockSpec((FUSE_BLK, 2 * EMBED_DIM), lambda i: (i, 0)),
        out_shape=jax.ShapeDtypeStruct((NUM_ENTITIES, 2 * EMBED_DIM),
                                       jnp.float32),
    )(re_t, im_t)


def _trig_body(rr_ref, ri_ref, out_ref):
    rr = rr_ref[...]
    out_ref[...] = jnp.concatenate(
        [jnp.cos(rr), jnp.sin(rr), jnp.cos(ri_ref[...]),
         jnp.zeros_like(rr)], axis=1)


def _relation_trig(relation_re, relation_im):
    return pl.pallas_call(
        _trig_body,
        out_shape=jax.ShapeDtypeStruct((NUM_RELATIONS, 4 * EMBED_DIM),
                                       jnp.float32),
    )(relation_re, relation_im)


def _sc_score_kernel(h_hbm, t_hbm, r_hbm, ent_hbm, rel_hbm, out_hbm,
                     idx_h, idx_t, idx_r, h_v, t_v, r_v, scores_v, sem):
    wid = lax.axis_index("s") * NC + lax.axis_index("c")
    base_w = wid * B_PER_W
    D = EMBED_DIM

    @pl.loop(0, N_CHUNKS)
    def _(chunk):
        base = base_w + chunk * CHUNK
        pltpu.sync_copy(h_hbm.at[pl.ds(base, CHUNK)], idx_h)
        pltpu.sync_copy(t_hbm.at[pl.ds(base, CHUNK)], idx_t)
        pltpu.sync_copy(r_hbm.at[pl.ds(base, CHUNK)], idx_r)
        copies = [
            pltpu.async_copy(ent_hbm.at[idx_h], h_v, sem),
            pltpu.async_copy(ent_hbm.at[idx_t], t_v, sem),
            pltpu.async_copy(rel_hbm.at[idx_r], r_v, sem),
        ]
        for cp_ in copies:
            cp_.wait()

        lane = lax.iota(jnp.int32, LANES)

        @pl.loop(0, CHUNK // LANES)
        def _(g):
            svec = jnp.zeros((LANES,), jnp.float32)
            for k in range(LANES):
                b = g * LANES + k
                acc = None
                for j in range(D // LANES):
                    sl = pl.ds(j * LANES, LANES)
                    sl_im = pl.ds(D + j * LANES, LANES)
                    trv = t_v[b, sl]
                    tiv = t_v[b, sl_im]
                    cc = r_v[b, sl]
                    ss = r_v[b, sl_im]
                    ci = r_v[b, pl.ds(2 * D + j * LANES, LANES)]
                    rot_r = trv * cc - tiv * ss
                    rot_i = trv * ss + tiv * ci
                    dr = h_v[b, sl] - rot_r
                    di = h_v[b, sl_im] - rot_i
                    part = dr * dr + di * di
                    acc = part if acc is None else acc + part
                svec = jnp.where(lane == k, jnp.sum(acc), svec)
            scores_v[pl.ds(g * LANES, LANES)] = svec

        pltpu.sync_copy(scores_v, out_hbm.at[pl.ds(base, CHUNK)])


def _sc_score(h, t, r, ent_fused, rel_fused):
    mesh = plsc.VectorSubcoreMesh(core_axis_name="c", subcore_axis_name="s")
    cp = pltpu.CompilerParams()
    if "needs_layout_passes" in pltpu.CompilerParams.__dataclass_fields__:
        cp = dataclasses.replace(cp, needs_layout_passes=False)
    run = functools.partial(
        pl.kernel,
        mesh=mesh,
        compiler_params=cp,
        out_type=jax.ShapeDtypeStruct((BATCH,), jnp.float32),
        scratch_types=[
            pltpu.VMEM((CHUNK,), jnp.int32),
            pltpu.VMEM((CHUNK,), jnp.int32),
            pltpu.VMEM((CHUNK,), jnp.int32),
            pltpu.VMEM((CHUNK, 2 * EMBED_DIM), jnp.float32),
            pltpu.VMEM((CHUNK, 2 * EMBED_DIM), jnp.float32),
            pltpu.VMEM((CHUNK, 4 * EMBED_DIM), jnp.float32),
            pltpu.VMEM((CHUNK,), jnp.float32),
            pltpu.SemaphoreType.DMA,
        ],
    )(_sc_score_kernel)
    return run(h, t, r, ent_fused, rel_fused)


def kernel(h, r, t, entity_re, entity_im, relation_re, relation_im):
    h = h.astype(jnp.int32)
    r = r.astype(jnp.int32)
    t = t.astype(jnp.int32)
    ent_fused = _fused_entity_table(entity_re, entity_im)
    rel_fused = _relation_trig(relation_re, relation_im)
    return _sc_score(h, t, r, ent_fused, rel_fused)


# speedup vs baseline: 2.4214x; 1.0629x over previous
import dataclasses
import functools

import jax
import jax.numpy as jnp
from jax import lax
from jax.experimental import pallas as pl
from jax.experimental.pallas import tpu as pltpu
from jax.experimental.pallas import tpu_sc as plsc

NUM_ENTITIES = 1000000
NUM_RELATIONS = 1000
EMBED_DIM = 64
BATCH = 16384

NC = 2
NS = 16
NW = NC * NS
LANES = 16

B_PER_W = BATCH // NW
CHUNK = 128
N_CHUNKS = B_PER_W // CHUNK

FUSE_BLK = 16384


def _fuse_body(re_ref, im_ref, out_ref):
    out_ref[...] = jnp.concatenate(
        [jnp.transpose(re_ref[...]), jnp.transpose(im_ref[...])], axis=1)


def _fused_entity_table(entity_re, entity_im):
    re_t = jnp.transpose(entity_re)
    im_t = jnp.transpose(entity_im)
    grid = pl.cdiv(NUM_ENTITIES, FUSE_BLK)
    return pl.pallas_call(
        _fuse_body,
        grid=(grid,),
        in_specs=[
            pl.BlockSpec((EMBED_DIM, FUSE_BLK), lambda i: (0, i)),
            pl.BlockSpec((EMBED_DIM, FUSE_BLK), lambda i: (0, i)),
        ],
        out_specs=pl.BlockSpec((FUSE_BLK, 2 * EMBED_DIM), lambda i: (i, 0)),
        out_shape=jax.ShapeDtypeStruct((NUM_ENTITIES, 2 * EMBED_DIM),
                                       jnp.float32),
    )(re_t, im_t)


def _trig_body(rr_ref, ri_ref, out_ref):
    rr = rr_ref[...]
    out_ref[...] = jnp.concatenate(
        [jnp.cos(rr), jnp.sin(rr), jnp.cos(ri_ref[...]),
         jnp.zeros_like(rr)], axis=1)


def _relation_trig(relation_re, relation_im):
    return pl.pallas_call(
        _trig_body,
        out_shape=jax.ShapeDtypeStruct((NUM_RELATIONS, 4 * EMBED_DIM),
                                       jnp.float32),
    )(relation_re, relation_im)


def _sc_score_kernel(h_hbm, t_hbm, r_hbm, ent_hbm, rel_hbm, out_hbm,
                     idx_h, idx_t, idx_r, h_v, t_v, r_v, scores_v, sem):
    wid = lax.axis_index("s") * NC + lax.axis_index("c")
    base_w = wid * B_PER_W
    D = EMBED_DIM

    @pl.loop(0, N_CHUNKS)
    def _(chunk):
        base = base_w + chunk * CHUNK
        pltpu.sync_copy(h_hbm.at[pl.ds(base, CHUNK)], idx_h)
        pltpu.sync_copy(t_hbm.at[pl.ds(base, CHUNK)], idx_t)
        pltpu.sync_copy(r_hbm.at[pl.ds(base, CHUNK)], idx_r)
        copies = [
            pltpu.async_copy(ent_hbm.at[idx_h], h_v, sem),
            pltpu.async_copy(ent_hbm.at[idx_t], t_v, sem),
            pltpu.async_copy(rel_hbm.at[idx_r], r_v, sem),
        ]
        for cp_ in copies:
            cp_.wait()

        lane = lax.iota(jnp.int32, LANES)

        @pl.loop(0, CHUNK // LANES)
        def _(g):
            svec = jnp.zeros((LANES,), jnp.float32)
            for k in range(LANES):
                b = g * LANES + k
                acc = None
                for j in range(D // LANES):
                    sl = pl.ds(j * LANES, LANES)
                    sl_im = pl.ds(D + j * LANES, LANES)
                    trv = t_v[b, sl]
                    tiv = t_v[b, sl_im]
                    cc = r_v[b, sl]
                    ss = r_v[b, sl_im]
                    ci = r_v[b, pl.ds(2 * D + j * LANES, LANES)]
                    rot_r = trv * cc - tiv * ss
                    rot_i = trv * ss + tiv * ci
                    dr = h_v[b, sl] - rot_r
                    di = h_v[b, sl_im] - rot_i
                    part = dr * dr + di * di
                    acc = part if acc is None else acc + part
                svec = jnp.where(lane == k, jnp.sum(acc), svec)
            scores_v[pl.ds(g * LANES, LANES)] = svec

        pltpu.sync_copy(scores_v, out_hbm.at[pl.ds(base, CHUNK)])


def _sc_score(h, t, r, ent_fused, rel_fused):
    mesh = plsc.VectorSubcoreMesh(core_axis_name="c", subcore_axis_name="s")
    cp = pltpu.CompilerParams()
    if "needs_layout_passes" in pltpu.CompilerParams.__dataclass_fields__:
        cp = dataclasses.replace(cp, needs_layout_passes=False)
    run = functools.partial(
        pl.kernel,
        mesh=mesh,
        compiler_params=cp,
        out_type=jax.ShapeDtypeStruct((BATCH,), jnp.float32),
        scratch_types=[
            pltpu.VMEM((CHUNK,), jnp.int32),
            pltpu.VMEM((CHUNK,), jnp.int32),
            pltpu.VMEM((CHUNK,), jnp.int32),
            pltpu.VMEM((CHUNK, 2 * EMBED_DIM), jnp.float32),
            pltpu.VMEM((CHUNK, 2 * EMBED_DIM), jnp.float32),
            pltpu.VMEM((CHUNK, 4 * EMBED_DIM), jnp.float32),
            pltpu.VMEM((CHUNK,), jnp.float32),
            pltpu.SemaphoreType.DMA,
        ],
    )(_sc_score_kernel)
    return run(h, t, r, ent_fused, rel_fused)


def kernel(h, r, t, entity_re, entity_im, relation_re, relation_im):
    h = h.astype(jnp.int32)
    r = r.astype(jnp.int32)
    t = t.astype(jnp.int32)
    ent_fused = _fused_entity_table(entity_re, entity_im)
    rel_fused = _relation_trig(relation_re, relation_im)
    return _sc_score(h, t, r, ent_fused, rel_fused)


# speedup vs baseline: 2.4566x; 1.0145x over previous
import dataclasses
import functools

import jax
import jax.numpy as jnp
from jax import lax
from jax.experimental import pallas as pl
from jax.experimental.pallas import tpu as pltpu
from jax.experimental.pallas import tpu_sc as plsc

NUM_ENTITIES = 1000000
NUM_RELATIONS = 1000
EMBED_DIM = 64
BATCH = 16384

NC = 2
NS = 16
NW = NC * NS
LANES = 16

B_PER_W = BATCH // NW
CHUNK = 64
N_CHUNKS = B_PER_W // CHUNK

FUSE_BLK = 16384


def _fuse_body(re_ref, im_ref, out_ref):
    out_ref[...] = jnp.concatenate(
        [jnp.transpose(re_ref[...]), jnp.transpose(im_ref[...])], axis=1)


def _fused_entity_table(entity_re, entity_im):
    re_t = jnp.transpose(entity_re)
    im_t = jnp.transpose(entity_im)
    grid = pl.cdiv(NUM_ENTITIES, FUSE_BLK)
    return pl.pallas_call(
        _fuse_body,
        grid=(grid,),
        in_specs=[
            pl.BlockSpec((EMBED_DIM, FUSE_BLK), lambda i: (0, i)),
            pl.BlockSpec((EMBED_DIM, FUSE_BLK), lambda i: (0, i)),
        ],
        out_specs=pl.BlockSpec((FUSE_BLK, 2 * EMBED_DIM), lambda i: (i, 0)),
        out_shape=jax.ShapeDtypeStruct((NUM_ENTITIES, 2 * EMBED_DIM),
                                       jnp.float32),
    )(re_t, im_t)


def _trig_body(rr_ref, ri_ref, out_ref):
    rr = rr_ref[...]
    out_ref[...] = jnp.concatenate(
        [jnp.cos(rr), jnp.sin(rr), jnp.cos(ri_ref[...]),
         jnp.zeros_like(rr)], axis=1)


def _relation_trig(relation_re, relation_im):
    return pl.pallas_call(
        _trig_body,
        out_shape=jax.ShapeDtypeStruct((NUM_RELATIONS, 4 * EMBED_DIM),
                                       jnp.float32),
    )(relation_re, relation_im)


def _sc_score_kernel(h_hbm, t_hbm, r_hbm, ent_hbm, rel_hbm, out_hbm,
                     idx_h0, idx_t0, idx_r0, h_v0, t_v0, r_v0,
                     idx_h1, idx_t1, idx_r1, h_v1, t_v1, r_v1,
                     scores_v, sem0, sem1):
    wid = lax.axis_index("s") * NC + lax.axis_index("c")
    base_w = wid * B_PER_W
    D = EMBED_DIM
    bufs = ((idx_h0, idx_t0, idx_r0, h_v0, t_v0, r_v0, sem0),
            (idx_h1, idx_t1, idx_r1, h_v1, t_v1, r_v1, sem1))

    def fire(chunk, buf):
        idx_h, idx_t, idx_r, h_v, t_v, r_v, sem = buf
        base = base_w + chunk * CHUNK
        pltpu.sync_copy(h_hbm.at[pl.ds(base, CHUNK)], idx_h)
        pltpu.sync_copy(t_hbm.at[pl.ds(base, CHUNK)], idx_t)
        pltpu.sync_copy(r_hbm.at[pl.ds(base, CHUNK)], idx_r)
        return [
            pltpu.async_copy(ent_hbm.at[idx_h], h_v, sem),
            pltpu.async_copy(ent_hbm.at[idx_t], t_v, sem),
            pltpu.async_copy(rel_hbm.at[idx_r], r_v, sem),
        ]

    def compute(chunk, buf):
        _, _, _, h_v, t_v, r_v, _ = buf
        base = base_w + chunk * CHUNK
        lane = lax.iota(jnp.int32, LANES)

        @pl.loop(0, CHUNK // LANES)
        def _(g):
            svec = jnp.zeros((LANES,), jnp.float32)
            for k in range(LANES):
                b = g * LANES + k
                acc = None
                for j in range(D // LANES):
                    sl = pl.ds(j * LANES, LANES)
                    sl_im = pl.ds(D + j * LANES, LANES)
                    trv = t_v[b, sl]
                    tiv = t_v[b, sl_im]
                    cc = r_v[b, sl]
                    ss = r_v[b, sl_im]
                    ci = r_v[b, pl.ds(2 * D + j * LANES, LANES)]
                    rot_r = trv * cc - tiv * ss
                    rot_i = trv * ss + tiv * ci
                    dr = h_v[b, sl] - rot_r
                    di = h_v[b, sl_im] - rot_i
                    part = dr * dr + di * di
                    acc = part if acc is None else acc + part
                svec = jnp.where(lane == k, jnp.sum(acc), svec)
            scores_v[pl.ds(g * LANES, LANES)] = svec

        pltpu.sync_copy(scores_v, out_hbm.at[pl.ds(base, CHUNK)])

    def wait3(buf):
        idx_h, idx_t, idx_r, h_v, t_v, r_v, sem = buf
        pltpu.make_async_copy(ent_hbm.at[idx_h], h_v, sem).wait()
        pltpu.make_async_copy(ent_hbm.at[idx_t], t_v, sem).wait()
        pltpu.make_async_copy(rel_hbm.at[idx_r], r_v, sem).wait()

    fire(0, bufs[0])

    @pl.loop(0, N_CHUNKS, step=2)
    def _(c):
        fire(c + 1, bufs[1])
        wait3(bufs[0])
        compute(c, bufs[0])

        @pl.when(c + 2 < N_CHUNKS)
        def _():
            fire(c + 2, bufs[0])

        wait3(bufs[1])
        compute(c + 1, bufs[1])


def _sc_score(h, t, r, ent_fused, rel_fused):
    mesh = plsc.VectorSubcoreMesh(core_axis_name="c", subcore_axis_name="s")
    cp = pltpu.CompilerParams()
    if "needs_layout_passes" in pltpu.CompilerParams.__dataclass_fields__:
        cp = dataclasses.replace(cp, needs_layout_passes=False)
    run = functools.partial(
        pl.kernel,
        mesh=mesh,
        compiler_params=cp,
        out_type=jax.ShapeDtypeStruct((BATCH,), jnp.float32),
        scratch_types=[
            pltpu.VMEM((CHUNK,), jnp.int32),
            pltpu.VMEM((CHUNK,), jnp.int32),
            pltpu.VMEM((CHUNK,), jnp.int32),
            pltpu.VMEM((CHUNK, 2 * EMBED_DIM), jnp.float32),
            pltpu.VMEM((CHUNK, 2 * EMBED_DIM), jnp.float32),
            pltpu.VMEM((CHUNK, 4 * EMBED_DIM), jnp.float32),
            pltpu.VMEM((CHUNK,), jnp.int32),
            pltpu.VMEM((CHUNK,), jnp.int32),
            pltpu.VMEM((CHUNK,), jnp.int32),
            pltpu.VMEM((CHUNK, 2 * EMBED_DIM), jnp.float32),
            pltpu.VMEM((CHUNK, 2 * EMBED_DIM), jnp.float32),
            pltpu.VMEM((CHUNK, 4 * EMBED_DIM), jnp.float32),
            pltpu.VMEM((CHUNK,), jnp.float32),
            pltpu.SemaphoreType.DMA,
            pltpu.SemaphoreType.DMA,
        ],
    )(_sc_score_kernel)
    return run(h, t, r, ent_fused, rel_fused)


def kernel(h, r, t, entity_re, entity_im, relation_re, relation_im):
    h = h.astype(jnp.int32)
    r = r.astype(jnp.int32)
    t = t.astype(jnp.int32)
    ent_fused = _fused_entity_table(entity_re, entity_im)
    rel_fused = _relation_trig(relation_re, relation_im)
    return _sc_score(h, t, r, ent_fused, rel_fused)
